# baseline (speedup 1.0000x reference)
import functools

import jax
import jax.numpy as jnp
from jax import lax
from jax.experimental import pallas as pl
from jax.experimental.pallas import tpu as pltpu
from jax.experimental.pallas import tpu_sc as plsc

N = 10000
NP = 10240
NFEAT = 128
NHID = 16
HEADS = 8
E = 320000
EL = E + N
C = 108
NW = 32
CHB = 32
NBLK = 3
CH = CHB * NBLK
EP = NW * C * CH
ROWS_PT = NP // 16
W1CAT = 136
W2CAT = 32


def _proj_body(x_ref, wa_ref, wb_ref, oa_ref, ob_ref):
    x = x_ref[...]
    oa_ref[...] = jnp.dot(x, wa_ref[...], preferred_element_type=jnp.float32)
    ob_ref[...] = jnp.dot(x, wb_ref[...], preferred_element_type=jnp.float32)


def _dense_project(x_pad, wa, wb):
    bn = 1024
    k = x_pad.shape[1]
    return pl.pallas_call(
        _proj_body,
        grid=(NP // bn,),
        in_specs=[
            pl.BlockSpec((bn, k), lambda i: (i, 0)),
            pl.BlockSpec((k, wa.shape[1]), lambda i: (0, 0)),
            pl.BlockSpec((k, wb.shape[1]), lambda i: (0, 0)),
        ],
        out_specs=[
            pl.BlockSpec((bn, wa.shape[1]), lambda i: (i, 0)),
            pl.BlockSpec((bn, wb.shape[1]), lambda i: (i, 0)),
        ],
        out_shape=[
            jax.ShapeDtypeStruct((NP, wa.shape[1]), jnp.float32),
            jax.ShapeDtypeStruct((NP, wb.shape[1]), jnp.float32),
        ],
    )(x_pad, wa, wb)


def _combine1_body(pa_ref, pb_ref, b1_ref, r_ref, w2e_ref, w2d_ref, t2_ref, t2d_ref):
    acc = pa_ref[...] + pb_ref[...]
    out128 = acc[:, :128]
    den = acc[:, 128:136]
    recip = 1.0 / (den + 1e-16)
    rec128 = jnp.dot(recip, r_ref[...], preferred_element_type=jnp.float32)
    x2 = jnp.maximum(out128 * rec128 + b1_ref[...], 0.0)
    t2_ref[...] = jnp.dot(x2, w2e_ref[...], preferred_element_type=jnp.float32)
    t2d_ref[...] = jnp.dot(x2, w2d_ref[...], preferred_element_type=jnp.float32)


def _combine1(pa, pb, b1, r, w2e, w2d):
    bn = 1024
    return pl.pallas_call(
        _combine1_body,
        grid=(NP // bn,),
        in_specs=[
            pl.BlockSpec((bn, W1CAT), lambda i: (i, 0)),
            pl.BlockSpec((bn, W1CAT), lambda i: (i, 0)),
            pl.BlockSpec((1, 128), lambda i: (0, 0)),
            pl.BlockSpec((8, 128), lambda i: (0, 0)),
            pl.BlockSpec((128, W2CAT), lambda i: (0, 0)),
            pl.BlockSpec((128, 16), lambda i: (0, 0)),
        ],
        out_specs=[
            pl.BlockSpec((bn, W2CAT), lambda i: (i, 0)),
            pl.BlockSpec((bn, 16), lambda i: (i, 0)),
        ],
        out_shape=[
            jax.ShapeDtypeStruct((NP, W2CAT), jnp.float32),
            jax.ShapeDtypeStruct((NP, 16), jnp.float32),
        ],
    )(pa, pb, b1, r, w2e, w2d)


def _combine2_body(pa_ref, pb_ref, b2_ref, o_ref):
    acc = pa_ref[...] + pb_ref[...]
    num = acc[:, :16]
    recip = 1.0 / (acc[:, 16:17] + 1e-16)
    rec16 = jnp.dot(recip, jnp.ones((1, 16), jnp.float32),
                    preferred_element_type=jnp.float32)
    o_ref[...] = jnp.maximum(num * rec16 + b2_ref[...], 0.0)


def _combine2(pa, pb, b2):
    bn = 1024
    return pl.pallas_call(
        _combine2_body,
        grid=(NP // bn,),
        in_specs=[
            pl.BlockSpec((bn, W2CAT), lambda i: (i, 0)),
            pl.BlockSpec((bn, W2CAT), lambda i: (i, 0)),
            pl.BlockSpec((1, 16), lambda i: (0, 0)),
        ],
        out_specs=pl.BlockSpec((bn, 16), lambda i: (i, 0)),
        out_shape=jax.ShapeDtypeStruct((NP, 16), jnp.float32),
    )(pa, pb, b2)


def _sc_edge_kernel(width, nheads):
    feat = nheads * 16
    att_off = width - 16
    ov = feat - att_off
    mesh = plsc.VectorSubcoreMesh(core_axis_name="c", subcore_axis_name="s",
                                  num_cores=2, num_subcores=16)

    def body(tsrc_hbm, tdst_hbm, src_hbm, dst_hbm, zro_hbm, out_hbm,
             idx_s, idx_d, rows_a, rows_b, arows_a, arows_b, acc,
             sem_a, sem_b, wsem_a, wsem_b):
        cid = lax.axis_index("c")
        sid = lax.axis_index("s")
        wid = cid * 16 + sid
        r0 = sid * ROWS_PT
        pltpu.sync_copy(zro_hbm, acc.at[pl.ds(r0, ROWS_PT)])
        plsc.subcore_barrier()

        lane = lax.iota(jnp.int32, 16)
        att_sl = pl.ds(att_off, 16)

        def make_edge(rows, arows):
            def edge(e, _):
                va = rows[e, att_sl]
                vb = arows[e, pl.ds(0, 16)]
                t = va + vb
                al = jnp.where(t >= 0.0, t, t * 0.2)
                w = jnp.exp(al)
                for h in range(nheads):
                    sl = pl.ds(h * 16, 16)
                    rows[e, sl] = rows[e, sl] * w[ov + h]
                wm = jnp.where(lane < ov + nheads, w, 0.0)
                if ov > 0:
                    cur = rows[e, att_sl]
                    wm = jnp.where(lane < ov, cur, wm)
                rows[e, att_sl] = wm
                return 0
            return edge

        def issue(c, rbuf, abuf, sem):
            pltpu.async_copy(tsrc_hbm.at[idx_s.at[c]], rbuf, sem)
            pltpu.async_copy(tdst_hbm.at[idx_d.at[c]], abuf, sem)

        def process(c, rbuf, abuf, sem, wsem):
            pltpu.make_async_copy(tsrc_hbm.at[idx_s.at[0]], rbuf, sem).wait()
            pltpu.make_async_copy(tdst_hbm.at[idx_d.at[0]], abuf, sem).wait()
            lax.fori_loop(0, C, make_edge(rbuf, abuf), 0)
            pltpu.async_copy(rbuf, acc.at[idx_d.at[c]], wsem, add=True)

        def wait_scatter(rbuf, wsem):
            pltpu.make_async_copy(rbuf, acc.at[idx_d.at[0]], wsem).wait()

        def pair(j, _):
            c = 2 * j
            process(c, rows_a, arows_a, sem_a, wsem_a)
            process(c + 1, rows_b, arows_b, sem_b, wsem_b)

            @pl.when(j < CHB // 2 - 1)
            def _():
                wait_scatter(rows_a, wsem_a)
                issue(c + 2, rows_a, arows_a, sem_a)
                wait_scatter(rows_b, wsem_b)
                issue(c + 3, rows_b, arows_b, sem_b)

            return 0

        def blk(b, _):
            pltpu.sync_copy(src_hbm.at[wid, pl.ds(b * CHB, CHB)], idx_s)
            pltpu.sync_copy(dst_hbm.at[wid, pl.ds(b * CHB, CHB)], idx_d)
            issue(0, rows_a, arows_a, sem_a)
            issue(1, rows_b, arows_b, sem_b)
            lax.fori_loop(0, CHB // 2, pair, 0)
            wait_scatter(rows_a, wsem_a)
            wait_scatter(rows_b, wsem_b)
            return 0

        lax.fori_loop(0, NBLK, blk, 0)
        plsc.subcore_barrier()
        pltpu.sync_copy(acc.at[pl.ds(r0, ROWS_PT)],
                        out_hbm.at[cid, pl.ds(r0, ROWS_PT)])

    return pl.kernel(
        body,
        out_type=jax.ShapeDtypeStruct((2, NP, width), jnp.float32),
        mesh=mesh,
        compiler_params=pltpu.CompilerParams(use_tc_tiling_on_sc=False),
        scratch_types=[
            pltpu.VMEM((CHB, C), jnp.int32),
            pltpu.VMEM((CHB, C), jnp.int32),
            pltpu.VMEM((C, width), jnp.float32),
            pltpu.VMEM((C, width), jnp.float32),
            pltpu.VMEM((C, 16), jnp.float32),
            pltpu.VMEM((C, 16), jnp.float32),
            pltpu.VMEM_SHARED((NP, width), jnp.float32),
            pltpu.SemaphoreType.DMA,
            pltpu.SemaphoreType.DMA,
            pltpu.SemaphoreType.DMA,
            pltpu.SemaphoreType.DMA,
        ],
    )


def kernel(x, edge_index, W1, att_src1, att_dst1, b1, W2, att_src2, att_dst2, b2):
    f32 = jnp.float32
    x = x.astype(f32)
    ei = edge_index.astype(jnp.int32)
    loop = jnp.arange(N, dtype=jnp.int32)
    pad = jnp.full((EP - EL,), N, jnp.int32)
    srcp = jnp.concatenate([ei[0], loop, pad]).reshape(NW, CH, C)
    dstp = jnp.concatenate([ei[1], loop, pad]).reshape(NW, CH, C)
    x_pad = jnp.pad(x, ((0, NP - N), (0, 0)))

    a1s = att_src1.reshape(HEADS, NHID).astype(f32)
    a1d = att_dst1.reshape(HEADS, NHID).astype(f32)
    eye8 = jnp.eye(HEADS, dtype=f32)
    asrc = (eye8[:, None, :] * a1s[:, :, None]).reshape(HEADS * NHID, HEADS)
    adst = (eye8[:, None, :] * a1d[:, :, None]).reshape(HEADS * NHID, HEADS)
    w1 = W1.astype(f32)
    w1e = jnp.concatenate([w1, w1 @ asrc], axis=1)
    w1d = jnp.concatenate([jnp.zeros((NFEAT, 8), f32), w1 @ adst], axis=1)

    t1, ta = _dense_project(x_pad, w1e, w1d)
    z1 = jnp.zeros((ROWS_PT, W1CAT), f32)
    part1 = _sc_edge_kernel(W1CAT, HEADS)(t1, ta, srcp, dstp, z1)

    a2s = att_src2.reshape(NHID, 1).astype(f32)
    a2d = att_dst2.reshape(NHID, 1).astype(f32)
    w2 = W2.astype(f32)
    w2e = jnp.concatenate([w2, w2 @ a2s, jnp.zeros((HEADS * NHID, 15), f32)], axis=1)
    w2d = jnp.concatenate([w2 @ a2d, jnp.zeros((HEADS * NHID, 15), f32)], axis=1)
    rexp = jnp.repeat(eye8, NHID, axis=1)

    t2, t2d = _combine1(part1[0], part1[1], b1.reshape(1, 128).astype(f32),
                        rexp, w2e, w2d)
    z2 = jnp.zeros((ROWS_PT, W2CAT), f32)
    part2 = _sc_edge_kernel(W2CAT, 1)(t2, t2d, srcp, dstp, z2)

    out = _combine2(part2[0], part2[1], b2.reshape(1, 16).astype(f32))
    return out[:N]


# speedup vs baseline: 1.0679x; 1.0679x over previous
import functools

import jax
import jax.numpy as jnp
from jax import lax
from jax.experimental import pallas as pl
from jax.experimental.pallas import tpu as pltpu
from jax.experimental.pallas import tpu_sc as plsc

N = 10000
NP = 10240
NFEAT = 128
NHID = 16
HEADS = 8
E = 320000
EL = E + N
C = 108
NW = 32
CHB = 32
NBLK = 3
CH = CHB * NBLK
EP = NW * C * CH
ROWS_PT = NP // 16
W1CAT = 136
W2CAT = 32


def _proj_body(x_ref, wa_ref, wb_ref, oa_ref, ob_ref):
    x = x_ref[...]
    oa_ref[...] = jnp.dot(x, wa_ref[...], preferred_element_type=jnp.float32)
    ob_ref[...] = jnp.dot(x, wb_ref[...], preferred_element_type=jnp.float32)


def _dense_project(x_pad, wa, wb):
    bn = 1024
    k = x_pad.shape[1]
    return pl.pallas_call(
        _proj_body,
        grid=(NP // bn,),
        in_specs=[
            pl.BlockSpec((bn, k), lambda i: (i, 0)),
            pl.BlockSpec((k, wa.shape[1]), lambda i: (0, 0)),
            pl.BlockSpec((k, wb.shape[1]), lambda i: (0, 0)),
        ],
        out_specs=[
            pl.BlockSpec((bn, wa.shape[1]), lambda i: (i, 0)),
            pl.BlockSpec((bn, wb.shape[1]), lambda i: (i, 0)),
        ],
        out_shape=[
            jax.ShapeDtypeStruct((NP, wa.shape[1]), jnp.float32),
            jax.ShapeDtypeStruct((NP, wb.shape[1]), jnp.float32),
        ],
    )(x_pad, wa, wb)


def _combine1_body(pa_ref, pb_ref, b1_ref, r_ref, w2e_ref, w2d_ref, t2_ref, t2d_ref):
    acc = pa_ref[...] + pb_ref[...]
    out128 = acc[:, :128]
    den = acc[:, 128:136]
    recip = 1.0 / (den + 1e-16)
    rec128 = jnp.dot(recip, r_ref[...], preferred_element_type=jnp.float32)
    x2 = jnp.maximum(out128 * rec128 + b1_ref[...], 0.0)
    t2_ref[...] = jnp.dot(x2, w2e_ref[...], preferred_element_type=jnp.float32)
    t2d_ref[...] = jnp.dot(x2, w2d_ref[...], preferred_element_type=jnp.float32)


def _combine1(pa, pb, b1, r, w2e, w2d):
    bn = 1024
    return pl.pallas_call(
        _combine1_body,
        grid=(NP // bn,),
        in_specs=[
            pl.BlockSpec((bn, W1CAT), lambda i: (i, 0)),
            pl.BlockSpec((bn, W1CAT), lambda i: (i, 0)),
            pl.BlockSpec((1, 128), lambda i: (0, 0)),
            pl.BlockSpec((8, 128), lambda i: (0, 0)),
            pl.BlockSpec((128, W2CAT), lambda i: (0, 0)),
            pl.BlockSpec((128, 16), lambda i: (0, 0)),
        ],
        out_specs=[
            pl.BlockSpec((bn, W2CAT), lambda i: (i, 0)),
            pl.BlockSpec((bn, 16), lambda i: (i, 0)),
        ],
        out_shape=[
            jax.ShapeDtypeStruct((NP, W2CAT), jnp.float32),
            jax.ShapeDtypeStruct((NP, 16), jnp.float32),
        ],
    )(pa, pb, b1, r, w2e, w2d)


def _combine2_body(pa_ref, pb_ref, b2_ref, o_ref):
    acc = pa_ref[...] + pb_ref[...]
    num = acc[:, :16]
    recip = 1.0 / (acc[:, 16:17] + 1e-16)
    rec16 = jnp.dot(recip, jnp.ones((1, 16), jnp.float32),
                    preferred_element_type=jnp.float32)
    o_ref[...] = jnp.maximum(num * rec16 + b2_ref[...], 0.0)


def _combine2(pa, pb, b2):
    bn = 1024
    return pl.pallas_call(
        _combine2_body,
        grid=(NP // bn,),
        in_specs=[
            pl.BlockSpec((bn, W2CAT), lambda i: (i, 0)),
            pl.BlockSpec((bn, W2CAT), lambda i: (i, 0)),
            pl.BlockSpec((1, 16), lambda i: (0, 0)),
        ],
        out_specs=pl.BlockSpec((bn, 16), lambda i: (i, 0)),
        out_shape=jax.ShapeDtypeStruct((NP, 16), jnp.float32),
    )(pa, pb, b2)


def _sc_edge_kernel(width, nheads, stage=False):
    feat = nheads * 16
    att_off = width - 16
    ov = feat - att_off
    mesh = plsc.VectorSubcoreMesh(core_axis_name="c", subcore_axis_name="s",
                                  num_cores=2, num_subcores=16)

    def body(tsrc_hbm, tdst_hbm, src_hbm, dst_hbm, zro_hbm, out_hbm,
             idx_s, idx_d, rows_a, rows_b, arows_a, arows_b, acc, *rest):
        if stage:
            tsp, tdp, sem_a, sem_b = rest
            gsrc, gdst = tsp, tdp
        else:
            sem_a, sem_b = rest
            gsrc, gdst = tsrc_hbm, tdst_hbm
        cid = lax.axis_index("c")
        sid = lax.axis_index("s")
        wid = cid * 16 + sid
        r0 = sid * ROWS_PT
        pltpu.sync_copy(zro_hbm, acc.at[pl.ds(r0, ROWS_PT)])
        if stage:
            pltpu.sync_copy(tsrc_hbm.at[pl.ds(r0, ROWS_PT)], tsp.at[pl.ds(r0, ROWS_PT)])
            pltpu.sync_copy(tdst_hbm.at[pl.ds(r0, ROWS_PT)], tdp.at[pl.ds(r0, ROWS_PT)])
        plsc.subcore_barrier()

        lane = lax.iota(jnp.int32, 16)
        att_sl = pl.ds(att_off, 16)

        def make_edge(rows, arows):
            def edge(e, _):
                va = rows[e, att_sl]
                vb = arows[e, pl.ds(0, 16)]
                t = va + vb
                al = jnp.where(t >= 0.0, t, t * 0.2)
                w = jnp.exp(al)
                for h in range(nheads):
                    sl = pl.ds(h * 16, 16)
                    rows[e, sl] = rows[e, sl] * w[ov + h]
                wm = jnp.where(lane < ov + nheads, w, 0.0)
                if ov > 0:
                    cur = rows[e, att_sl]
                    wm = jnp.where(lane < ov, cur, wm)
                rows[e, att_sl] = wm
                return 0
            return edge

        def issue(c, rbuf, abuf, sem):
            pltpu.async_copy(gsrc.at[idx_s.at[c]], rbuf, sem)
            pltpu.async_copy(gdst.at[idx_d.at[c]], abuf, sem)

        def process(c, rbuf, abuf, sem):
            pltpu.make_async_copy(gsrc.at[idx_s.at[0]], rbuf, sem).wait()
            pltpu.make_async_copy(gdst.at[idx_d.at[0]], abuf, sem).wait()
            lax.fori_loop(0, C, make_edge(rbuf, abuf), 0)
            pltpu.sync_copy(rbuf, acc.at[idx_d.at[c]], add=True)

        def pair(j, _):
            c = 2 * j
            issue(c + 1, rows_b, arows_b, sem_b)
            process(c, rows_a, arows_a, sem_a)

            @pl.when(j < CHB // 2 - 1)
            def _():
                issue(c + 2, rows_a, arows_a, sem_a)

            process(c + 1, rows_b, arows_b, sem_b)
            return 0

        def blk(b, _):
            pltpu.sync_copy(src_hbm.at[wid, pl.ds(b * CHB, CHB)], idx_s)
            pltpu.sync_copy(dst_hbm.at[wid, pl.ds(b * CHB, CHB)], idx_d)
            issue(0, rows_a, arows_a, sem_a)
            lax.fori_loop(0, CHB // 2, pair, 0)
            return 0

        lax.fori_loop(0, NBLK, blk, 0)
        plsc.subcore_barrier()
        pltpu.sync_copy(acc.at[pl.ds(r0, ROWS_PT)],
                        out_hbm.at[cid, pl.ds(r0, ROWS_PT)])

    return pl.kernel(
        body,
        out_type=jax.ShapeDtypeStruct((2, NP, width), jnp.float32),
        mesh=mesh,
        compiler_params=pltpu.CompilerParams(use_tc_tiling_on_sc=False),
        scratch_types=[
            pltpu.VMEM((CHB, C), jnp.int32),
            pltpu.VMEM((CHB, C), jnp.int32),
            pltpu.VMEM((C, width), jnp.float32),
            pltpu.VMEM((C, width), jnp.float32),
            pltpu.VMEM((C, 16), jnp.float32),
            pltpu.VMEM((C, 16), jnp.float32),
            pltpu.VMEM_SHARED((NP, width), jnp.float32),
        ] + ([pltpu.VMEM_SHARED((NP, width), jnp.float32),
              pltpu.VMEM_SHARED((NP, 16), jnp.float32)] if stage else []) + [
            pltpu.SemaphoreType.DMA,
            pltpu.SemaphoreType.DMA,
        ],
    )


def kernel(x, edge_index, W1, att_src1, att_dst1, b1, W2, att_src2, att_dst2, b2):
    f32 = jnp.float32
    x = x.astype(f32)
    ei = edge_index.astype(jnp.int32)
    loop = jnp.arange(N, dtype=jnp.int32)
    pad = jnp.full((EP - EL,), N, jnp.int32)
    srcp = jnp.concatenate([ei[0], loop, pad]).reshape(NW, CH, C)
    dstp = jnp.concatenate([ei[1], loop, pad]).reshape(NW, CH, C)
    x_pad = jnp.pad(x, ((0, NP - N), (0, 0)))

    a1s = att_src1.reshape(HEADS, NHID).astype(f32)
    a1d = att_dst1.reshape(HEADS, NHID).astype(f32)
    eye8 = jnp.eye(HEADS, dtype=f32)
    asrc = (eye8[:, None, :] * a1s[:, :, None]).reshape(HEADS * NHID, HEADS)
    adst = (eye8[:, None, :] * a1d[:, :, None]).reshape(HEADS * NHID, HEADS)
    w1 = W1.astype(f32)
    w1e = jnp.concatenate([w1, w1 @ asrc], axis=1)
    w1d = jnp.concatenate([jnp.zeros((NFEAT, 8), f32), w1 @ adst], axis=1)

    t1, ta = _dense_project(x_pad, w1e, w1d)
    z1 = jnp.zeros((ROWS_PT, W1CAT), f32)
    part1 = _sc_edge_kernel(W1CAT, HEADS)(t1, ta, srcp, dstp, z1)

    a2s = att_src2.reshape(NHID, 1).astype(f32)
    a2d = att_dst2.reshape(NHID, 1).astype(f32)
    w2 = W2.astype(f32)
    w2e = jnp.concatenate([w2, w2 @ a2s, jnp.zeros((HEADS * NHID, 15), f32)], axis=1)
    w2d = jnp.concatenate([w2 @ a2d, jnp.zeros((HEADS * NHID, 15), f32)], axis=1)
    rexp = jnp.repeat(eye8, NHID, axis=1)

    t2, t2d = _combine1(part1[0], part1[1], b1.reshape(1, 128).astype(f32),
                        rexp, w2e, w2d)
    z2 = jnp.zeros((ROWS_PT, W2CAT), f32)
    part2 = _sc_edge_kernel(W2CAT, 1, stage=True)(t2, t2d, srcp, dstp, z2)

    out = _combine2(part2[0], part2[1], b2.reshape(1, 16).astype(f32))
    return out[:N]


# speedup vs baseline: 1.0875x; 1.0183x over previous
import functools

import jax
import jax.numpy as jnp
from jax import lax
from jax.experimental import pallas as pl
from jax.experimental.pallas import tpu as pltpu
from jax.experimental.pallas import tpu_sc as plsc

N = 10000
NP = 10240
NFEAT = 128
NHID = 16
HEADS = 8
E = 320000
EL = E + N
C = 96
NW = 32
CHB = 36
NBLK = 3
CH = CHB * NBLK
EP = NW * C * CH
ROWS_PT = NP // 16
W1CAT = 144
W2CAT = 32


def _proj_body(x_ref, wa_ref, wb_ref, oa_ref, ob_ref):
    x = x_ref[...]
    oa_ref[...] = jnp.dot(x, wa_ref[...], preferred_element_type=jnp.float32)
    ob_ref[...] = jnp.dot(x, wb_ref[...], preferred_element_type=jnp.float32)


def _dense_project(x_pad, wa, wb):
    bn = 1024
    k = x_pad.shape[1]
    return pl.pallas_call(
        _proj_body,
        grid=(NP // bn,),
        in_specs=[
            pl.BlockSpec((bn, k), lambda i: (i, 0)),
            pl.BlockSpec((k, wa.shape[1]), lambda i: (0, 0)),
            pl.BlockSpec((k, wb.shape[1]), lambda i: (0, 0)),
        ],
        out_specs=[
            pl.BlockSpec((bn, wa.shape[1]), lambda i: (i, 0)),
            pl.BlockSpec((bn, wb.shape[1]), lambda i: (i, 0)),
        ],
        out_shape=[
            jax.ShapeDtypeStruct((NP, wa.shape[1]), jnp.float32),
            jax.ShapeDtypeStruct((NP, wb.shape[1]), jnp.float32),
        ],
    )(x_pad, wa, wb)


def _combine1_body(pa_ref, pb_ref, b1_ref, r_ref, w2e_ref, w2d_ref, t2_ref, t2d_ref):
    acc = pa_ref[...] + pb_ref[...]
    out128 = acc[:, :128]
    den = acc[:, 128:136]
    recip = 1.0 / (den + 1e-16)
    rec128 = jnp.dot(recip, r_ref[...], preferred_element_type=jnp.float32)
    x2 = jnp.maximum(out128 * rec128 + b1_ref[...], 0.0)
    t2_ref[...] = jnp.dot(x2, w2e_ref[...], preferred_element_type=jnp.float32)
    t2d_ref[...] = jnp.dot(x2, w2d_ref[...], preferred_element_type=jnp.float32)


def _combine1(pa, pb, b1, r, w2e, w2d):
    bn = 1024
    return pl.pallas_call(
        _combine1_body,
        grid=(NP // bn,),
        in_specs=[
            pl.BlockSpec((bn, W1CAT), lambda i: (i, 0)),
            pl.BlockSpec((bn, W1CAT), lambda i: (i, 0)),
            pl.BlockSpec((1, 128), lambda i: (0, 0)),
            pl.BlockSpec((8, 128), lambda i: (0, 0)),
            pl.BlockSpec((128, W2CAT), lambda i: (0, 0)),
            pl.BlockSpec((128, 16), lambda i: (0, 0)),
        ],
        out_specs=[
            pl.BlockSpec((bn, W2CAT), lambda i: (i, 0)),
            pl.BlockSpec((bn, 16), lambda i: (i, 0)),
        ],
        out_shape=[
            jax.ShapeDtypeStruct((NP, W2CAT), jnp.float32),
            jax.ShapeDtypeStruct((NP, 16), jnp.float32),
        ],
    )(pa, pb, b1, r, w2e, w2d)


def _combine2_body(pa_ref, pb_ref, b2_ref, o_ref):
    acc = pa_ref[...] + pb_ref[...]
    num = acc[:, :16]
    recip = 1.0 / (acc[:, 16:17] + 1e-16)
    rec16 = jnp.dot(recip, jnp.ones((1, 16), jnp.float32),
                    preferred_element_type=jnp.float32)
    o_ref[...] = jnp.maximum(num * rec16 + b2_ref[...], 0.0)


def _combine2(pa, pb, b2):
    bn = 1024
    return pl.pallas_call(
        _combine2_body,
        grid=(NP // bn,),
        in_specs=[
            pl.BlockSpec((bn, W2CAT), lambda i: (i, 0)),
            pl.BlockSpec((bn, W2CAT), lambda i: (i, 0)),
            pl.BlockSpec((1, 16), lambda i: (0, 0)),
        ],
        out_specs=pl.BlockSpec((bn, 16), lambda i: (i, 0)),
        out_shape=jax.ShapeDtypeStruct((NP, 16), jnp.float32),
    )(pa, pb, b2)


def _sc_edge_kernel(width, nheads, stage=False):
    feat = nheads * 16
    att_off = width - 16
    ov = feat - att_off
    mesh = plsc.VectorSubcoreMesh(core_axis_name="c", subcore_axis_name="s",
                                  num_cores=2, num_subcores=16)

    def body(tsrc_hbm, tdst_hbm, src_hbm, dst_hbm, zro_hbm, out_hbm,
             idx_s, idx_d, rows_a, rows_b, arows_a, arows_b, acc, *rest):
        if stage:
            tsp, tdp, sem_a, sem_b = rest
            gsrc, gdst = tsp, tdp
        else:
            sem_a, sem_b = rest
            gsrc, gdst = tsrc_hbm, tdst_hbm
        cid = lax.axis_index("c")
        sid = lax.axis_index("s")
        wid = cid * 16 + sid
        r0 = sid * ROWS_PT
        pltpu.sync_copy(zro_hbm, acc.at[pl.ds(r0, ROWS_PT)])
        if stage:
            pltpu.sync_copy(tsrc_hbm.at[pl.ds(r0, ROWS_PT)], tsp.at[pl.ds(r0, ROWS_PT)])
            pltpu.sync_copy(tdst_hbm.at[pl.ds(r0, ROWS_PT)], tdp.at[pl.ds(r0, ROWS_PT)])
        plsc.subcore_barrier()

        lane = lax.iota(jnp.int32, 16)
        att_sl = pl.ds(att_off, 16)

        def make_edge(rows, arows):
            def edge(e, _):
                va = rows[e, att_sl]
                vb = arows[e, pl.ds(0, 16)]
                t = va + vb
                al = jnp.where(t >= 0.0, t, t * 0.2)
                w = jnp.exp(al)
                for h in range(nheads):
                    sl = pl.ds(h * 16, 16)
                    rows[e, sl] = rows[e, sl] * w[ov + h]
                wm = jnp.where(lane < ov + nheads, w, 0.0)
                if ov > 0:
                    cur = rows[e, att_sl]
                    wm = jnp.where(lane < ov, cur, wm)
                rows[e, att_sl] = wm
                return 0
            return edge

        def issue(c, rbuf, abuf, sem):
            pltpu.async_copy(gsrc.at[idx_s.at[c]], rbuf, sem)
            pltpu.async_copy(gdst.at[idx_d.at[c]], abuf, sem)

        def process(c, rbuf, abuf, sem):
            pltpu.make_async_copy(gsrc.at[idx_s.at[0]], rbuf, sem).wait()
            pltpu.make_async_copy(gdst.at[idx_d.at[0]], abuf, sem).wait()
            lax.fori_loop(0, C, make_edge(rbuf, abuf), 0)
            pltpu.sync_copy(rbuf, acc.at[idx_d.at[c]], add=True)

        def pair(j, _):
            c = 2 * j
            issue(c + 1, rows_b, arows_b, sem_b)
            process(c, rows_a, arows_a, sem_a)

            @pl.when(j < CHB // 2 - 1)
            def _():
                issue(c + 2, rows_a, arows_a, sem_a)

            process(c + 1, rows_b, arows_b, sem_b)
            return 0

        def blk(b, _):
            pltpu.sync_copy(src_hbm.at[wid, pl.ds(b * CHB, CHB)], idx_s)
            pltpu.sync_copy(dst_hbm.at[wid, pl.ds(b * CHB, CHB)], idx_d)
            issue(0, rows_a, arows_a, sem_a)
            lax.fori_loop(0, CHB // 2, pair, 0)
            return 0

        lax.fori_loop(0, NBLK, blk, 0)
        plsc.subcore_barrier()
        pltpu.sync_copy(acc.at[pl.ds(r0, ROWS_PT)],
                        out_hbm.at[cid, pl.ds(r0, ROWS_PT)])

    return pl.kernel(
        body,
        out_type=jax.ShapeDtypeStruct((2, NP, width), jnp.float32),
        mesh=mesh,
        compiler_params=pltpu.CompilerParams(use_tc_tiling_on_sc=False),
        scratch_types=[
            pltpu.VMEM((CHB, C), jnp.int32),
            pltpu.VMEM((CHB, C), jnp.int32),
            pltpu.VMEM((C, width), jnp.float32),
            pltpu.VMEM((C, width), jnp.float32),
            pltpu.VMEM((C, 16), jnp.float32),
            pltpu.VMEM((C, 16), jnp.float32),
            pltpu.VMEM_SHARED((NP, width), jnp.float32),
        ] + ([pltpu.VMEM_SHARED((NP, width), jnp.float32),
              pltpu.VMEM_SHARED((NP, 16), jnp.float32)] if stage else []) + [
            pltpu.SemaphoreType.DMA,
            pltpu.SemaphoreType.DMA,
        ],
    )


def kernel(x, edge_index, W1, att_src1, att_dst1, b1, W2, att_src2, att_dst2, b2):
    f32 = jnp.float32
    x = x.astype(f32)
    ei = edge_index.astype(jnp.int32)
    loop = jnp.arange(N, dtype=jnp.int32)
    pad = jnp.full((EP - EL,), N, jnp.int32)
    srcp = jnp.concatenate([ei[0], loop, pad]).reshape(NW, CH, C)
    dstp = jnp.concatenate([ei[1], loop, pad]).reshape(NW, CH, C)
    x_pad = jnp.pad(x, ((0, NP - N), (0, 0)))

    a1s = att_src1.reshape(HEADS, NHID).astype(f32)
    a1d = att_dst1.reshape(HEADS, NHID).astype(f32)
    eye8 = jnp.eye(HEADS, dtype=f32)
    asrc = (eye8[:, None, :] * a1s[:, :, None]).reshape(HEADS * NHID, HEADS)
    adst = (eye8[:, None, :] * a1d[:, :, None]).reshape(HEADS * NHID, HEADS)
    w1 = W1.astype(f32)
    w1e = jnp.concatenate([w1, w1 @ asrc, jnp.zeros((NFEAT, 8), f32)], axis=1)
    w1d = jnp.concatenate([w1 @ adst, jnp.zeros((NFEAT, 8), f32)], axis=1)

    t1, ta = _dense_project(x_pad, w1e, w1d)
    z1 = jnp.zeros((ROWS_PT, W1CAT), f32)
    part1 = _sc_edge_kernel(W1CAT, HEADS)(t1, ta, srcp, dstp, z1)

    a2s = att_src2.reshape(NHID, 1).astype(f32)
    a2d = att_dst2.reshape(NHID, 1).astype(f32)
    w2 = W2.astype(f32)
    w2e = jnp.concatenate([w2, w2 @ a2s, jnp.zeros((HEADS * NHID, 15), f32)], axis=1)
    w2d = jnp.concatenate([w2 @ a2d, jnp.zeros((HEADS * NHID, 15), f32)], axis=1)
    rexp = jnp.repeat(eye8, NHID, axis=1)

    t2, t2d = _combine1(part1[0], part1[1], b1.reshape(1, 128).astype(f32),
                        rexp, w2e, w2d)
    z2 = jnp.zeros((ROWS_PT, W2CAT), f32)
    part2 = _sc_edge_kernel(W2CAT, 1)(t2, t2d, srcp, dstp, z2)

    out = _combine2(part2[0], part2[1], b2.reshape(1, 16).astype(f32))
    return out[:N]


# speedup vs baseline: 1.0883x; 1.0007x over previous
import functools

import jax
import jax.numpy as jnp
from jax import lax
from jax.experimental import pallas as pl
from jax.experimental.pallas import tpu as pltpu
from jax.experimental.pallas import tpu_sc as plsc

N = 10000
NP = 10240
NFEAT = 128
NHID = 16
HEADS = 8
E = 320000
EL = E + N
C = 96
NW = 32
CHB = 36
NBLK = 3
CH = CHB * NBLK
EP = NW * C * CH
ROWS_PT = NP // 16
W1CAT = 144
W2CAT = 32


def _proj_body(x_ref, wa_ref, wb_ref, oa_ref, ob_ref):
    x = x_ref[...]
    oa_ref[...] = jnp.dot(x, wa_ref[...], preferred_element_type=jnp.float32)
    ob_ref[...] = jnp.dot(x, wb_ref[...], preferred_element_type=jnp.float32)


def _dense_project(x_pad, wa, wb):
    bn = 1024
    k = x_pad.shape[1]
    return pl.pallas_call(
        _proj_body,
        grid=(NP // bn,),
        in_specs=[
            pl.BlockSpec((bn, k), lambda i: (i, 0)),
            pl.BlockSpec((k, wa.shape[1]), lambda i: (0, 0)),
            pl.BlockSpec((k, wb.shape[1]), lambda i: (0, 0)),
        ],
        out_specs=[
            pl.BlockSpec((bn, wa.shape[1]), lambda i: (i, 0)),
            pl.BlockSpec((bn, wb.shape[1]), lambda i: (i, 0)),
        ],
        out_shape=[
            jax.ShapeDtypeStruct((NP, wa.shape[1]), jnp.float32),
            jax.ShapeDtypeStruct((NP, wb.shape[1]), jnp.float32),
        ],
    )(x_pad, wa, wb)


def _combine1_body(pa_ref, pb_ref, b1_ref, r_ref, w2e_ref, w2d_ref, t2_ref, t2d_ref):
    acc = pa_ref[...] + pb_ref[...]
    out128 = acc[:, :128]
    den = acc[:, 128:136]
    recip = 1.0 / (den + 1e-16)
    rec128 = jnp.dot(recip, r_ref[...], preferred_element_type=jnp.float32)
    x2 = jnp.maximum(out128 * rec128 + b1_ref[...], 0.0)
    t2_ref[...] = jnp.dot(x2, w2e_ref[...], preferred_element_type=jnp.float32)
    t2d_ref[...] = jnp.dot(x2, w2d_ref[...], preferred_element_type=jnp.float32)


def _combine1(pa, pb, b1, r, w2e, w2d):
    bn = 1024
    return pl.pallas_call(
        _combine1_body,
        grid=(NP // bn,),
        in_specs=[
            pl.BlockSpec((bn, W1CAT), lambda i: (i, 0)),
            pl.BlockSpec((bn, W1CAT), lambda i: (i, 0)),
            pl.BlockSpec((1, 128), lambda i: (0, 0)),
            pl.BlockSpec((8, 128), lambda i: (0, 0)),
            pl.BlockSpec((128, W2CAT), lambda i: (0, 0)),
            pl.BlockSpec((128, 16), lambda i: (0, 0)),
        ],
        out_specs=[
            pl.BlockSpec((bn, W2CAT), lambda i: (i, 0)),
            pl.BlockSpec((bn, 16), lambda i: (i, 0)),
        ],
        out_shape=[
            jax.ShapeDtypeStruct((NP, W2CAT), jnp.float32),
            jax.ShapeDtypeStruct((NP, 16), jnp.float32),
        ],
    )(pa, pb, b1, r, w2e, w2d)


def _combine2_body(pa_ref, pb_ref, b2_ref, o_ref):
    acc = pa_ref[...] + pb_ref[...]
    num = acc[:, :16]
    recip = 1.0 / (acc[:, 16:17] + 1e-16)
    rec16 = jnp.dot(recip, jnp.ones((1, 16), jnp.float32),
                    preferred_element_type=jnp.float32)
    o_ref[...] = jnp.maximum(num * rec16 + b2_ref[...], 0.0)


def _combine2(pa, pb, b2):
    bn = 1024
    return pl.pallas_call(
        _combine2_body,
        grid=(NP // bn,),
        in_specs=[
            pl.BlockSpec((bn, W2CAT), lambda i: (i, 0)),
            pl.BlockSpec((bn, W2CAT), lambda i: (i, 0)),
            pl.BlockSpec((1, 16), lambda i: (0, 0)),
        ],
        out_specs=pl.BlockSpec((bn, 16), lambda i: (i, 0)),
        out_shape=jax.ShapeDtypeStruct((NP, 16), jnp.float32),
    )(pa, pb, b2)


def _sc_edge_kernel(width, nheads, stage=False):
    feat = nheads * 16
    att_off = width - 16
    ov = feat - att_off
    mesh = plsc.VectorSubcoreMesh(core_axis_name="c", subcore_axis_name="s",
                                  num_cores=2, num_subcores=16)

    def body(tsrc_hbm, tdst_hbm, src_hbm, dst_hbm, zro_hbm, out_hbm,
             idx_s, idx_d, rows_a, rows_b, arows_a, arows_b, acc, *rest):
        if stage:
            tsp, tdp, sem_a, sem_b = rest
            gsrc, gdst = tsp, tdp
        else:
            sem_a, sem_b = rest
            gsrc, gdst = tsrc_hbm, tdst_hbm
        cid = lax.axis_index("c")
        sid = lax.axis_index("s")
        wid = cid * 16 + sid
        r0 = sid * ROWS_PT
        pltpu.sync_copy(zro_hbm, acc.at[pl.ds(r0, ROWS_PT)])
        if stage:
            pltpu.sync_copy(tsrc_hbm.at[pl.ds(r0, ROWS_PT)], tsp.at[pl.ds(r0, ROWS_PT)])
            pltpu.sync_copy(tdst_hbm.at[pl.ds(r0, ROWS_PT)], tdp.at[pl.ds(r0, ROWS_PT)])
        plsc.subcore_barrier()

        lane = lax.iota(jnp.int32, 16)
        att_sl = pl.ds(att_off, 16)

        def make_edge(rows, arows):
            def edge(e, _):
                va = rows[e, att_sl]
                vb = arows[e, pl.ds(0, 16)]
                t = va + vb
                al = jnp.where(t >= 0.0, t, t * 0.2)
                w = jnp.exp(al)
                for h in range(nheads):
                    sl = pl.ds(h * 16, 16)
                    rows[e, sl] = rows[e, sl] * w[ov + h]
                wm = jnp.where(lane < ov + nheads, w, 0.0)
                if ov > 0:
                    cur = rows[e, att_sl]
                    wm = jnp.where(lane < ov, cur, wm)
                rows[e, att_sl] = wm
                return 0
            return edge

        def issue(c, rbuf, abuf, sem):
            pltpu.async_copy(gsrc.at[idx_s.at[c]], rbuf, sem)
            pltpu.async_copy(gdst.at[idx_d.at[c]], abuf, sem)

        def process(c, rbuf, abuf, sem):
            pltpu.make_async_copy(gsrc.at[idx_s.at[0]], rbuf, sem).wait()
            pltpu.make_async_copy(gdst.at[idx_d.at[0]], abuf, sem).wait()
            lax.fori_loop(0, C, make_edge(rbuf, abuf), 0)
            pltpu.sync_copy(rbuf, acc.at[idx_d.at[c]], add=True)

        def pair(j, _):
            c = 2 * j
            issue(c + 1, rows_b, arows_b, sem_b)
            process(c, rows_a, arows_a, sem_a)

            @pl.when(j < CHB // 2 - 1)
            def _():
                issue(c + 2, rows_a, arows_a, sem_a)

            process(c + 1, rows_b, arows_b, sem_b)
            return 0

        def blk(b, _):
            pltpu.sync_copy(src_hbm.at[wid, pl.ds(b * CHB, CHB)], idx_s)
            pltpu.sync_copy(dst_hbm.at[wid, pl.ds(b * CHB, CHB)], idx_d)
            issue(0, rows_a, arows_a, sem_a)
            lax.fori_loop(0, CHB // 2, pair, 0)
            return 0

        lax.fori_loop(0, NBLK, blk, 0)
        plsc.subcore_barrier()
        pltpu.sync_copy(acc.at[pl.ds(r0, ROWS_PT)],
                        out_hbm.at[cid, pl.ds(r0, ROWS_PT)])

    return pl.kernel(
        body,
        out_type=jax.ShapeDtypeStruct((2, NP, width), jnp.float32),
        mesh=mesh,
        compiler_params=pltpu.CompilerParams(use_tc_tiling_on_sc=False),
        scratch_types=[
            pltpu.VMEM((CHB, C), jnp.int32),
            pltpu.VMEM((CHB, C), jnp.int32),
            pltpu.VMEM((C, width), jnp.float32),
            pltpu.VMEM((C, width), jnp.float32),
            pltpu.VMEM((C, 16), jnp.float32),
            pltpu.VMEM((C, 16), jnp.float32),
            pltpu.VMEM_SHARED((NP, width), jnp.float32),
        ] + ([pltpu.VMEM_SHARED((NP, width), jnp.float32),
              pltpu.VMEM_SHARED((NP, 16), jnp.float32)] if stage else []) + [
            pltpu.SemaphoreType.DMA,
            pltpu.SemaphoreType.DMA,
        ],
    )


def kernel(x, edge_index, W1, att_src1, att_dst1, b1, W2, att_src2, att_dst2, b2):
    f32 = jnp.float32
    x = x.astype(f32)
    ei = edge_index.astype(jnp.int32)
    loop = jnp.arange(N, dtype=jnp.int32)
    pad = jnp.full((EP - EL,), N, jnp.int32)
    srcp = jnp.concatenate([ei[0], loop, pad]).reshape(NW, CH, C)
    dstp = jnp.concatenate([ei[1], loop, pad]).reshape(NW, CH, C)
    x_pad = jnp.pad(x, ((0, NP - N), (0, 0)))

    a1s = att_src1.reshape(HEADS, NHID).astype(f32)
    a1d = att_dst1.reshape(HEADS, NHID).astype(f32)
    eye8 = jnp.eye(HEADS, dtype=f32)
    asrc = (eye8[:, None, :] * a1s[:, :, None]).reshape(HEADS * NHID, HEADS)
    adst = (eye8[:, None, :] * a1d[:, :, None]).reshape(HEADS * NHID, HEADS)
    w1 = W1.astype(f32)
    w1e = jnp.concatenate([w1, w1 @ asrc, jnp.zeros((NFEAT, 8), f32)], axis=1)
    w1d = jnp.concatenate([w1 @ adst, jnp.zeros((NFEAT, 8), f32)], axis=1)

    t1, ta = _dense_project(x_pad, w1e, w1d)
    z1 = jnp.zeros((ROWS_PT, W1CAT), f32)
    part1 = _sc_edge_kernel(W1CAT, HEADS)(t1, ta, srcp, dstp, z1)

    a2s = att_src2.reshape(NHID, 1).astype(f32)
    a2d = att_dst2.reshape(NHID, 1).astype(f32)
    w2 = W2.astype(f32)
    w2e = jnp.concatenate([w2, w2 @ a2s, jnp.zeros((HEADS * NHID, 15), f32)], axis=1)
    w2d = jnp.concatenate([w2 @ a2d, jnp.zeros((HEADS * NHID, 15), f32)], axis=1)
    rexp = jnp.repeat(eye8, NHID, axis=1)

    t2, t2d = _combine1(part1[0], part1[1], b1.reshape(1, 128).astype(f32),
                        rexp, w2e, w2d)
    z2 = jnp.zeros((ROWS_PT, W2CAT), f32)
    part2 = _sc_edge_kernel(W2CAT, 1, stage=True)(t2, t2d, srcp, dstp, z2)

    out = _combine2(part2[0], part2[1], b2.reshape(1, 16).astype(f32))
    return out[:N]
